# Initial kernel scaffold; baseline (speedup 1.0000x reference)
import jax
import jax.numpy as jnp
from jax.experimental import pallas as pl


def kernel(input, target, enc_W, enc_b, codebook, dec_W, dec_b):
    raise NotImplementedError("write your pallas kernel here")


# speedup vs baseline: 1.2872x; 1.2872x over previous
import functools

import jax
import jax.numpy as jnp
from jax.experimental import pallas as pl
from jax.experimental.pallas import tpu as pltpu

P = 16
D = 32
M = 8192
TBLK = 128


def _vq_body(nblk, patches_ref, tpatch_ref, enc_w_ref, enc_b_ref, cbt_ref,
             cb_ref, dec_w_ref, dec_b_ref, dec_ref, rec_ref, vq_ref, loss_ref):
    i = pl.program_id(0)

    lat = jnp.dot(patches_ref[...], enc_w_ref[...],
                  preferred_element_type=jnp.float32) + enc_b_ref[...]
    scores = jnp.dot(lat, cbt_ref[...], preferred_element_type=jnp.float32)
    ind = jnp.argmax(scores, axis=1).astype(jnp.int32)

    onehot = (jax.lax.broadcasted_iota(jnp.int32, (TBLK, M), 1)
              == ind[:, None]).astype(jnp.float32)
    quant = jnp.dot(onehot, cb_ref[...], preferred_element_type=jnp.float32)

    num = jnp.sum(lat * quant, axis=1)
    ln = jnp.sqrt(jnp.sum(lat * lat, axis=1))
    qn = jnp.sqrt(jnp.sum(quant * quant, axis=1))
    cos = num / jnp.maximum(ln * qn, 1e-8)
    vq_blk = jnp.sum(1.0 - cos)

    dec = jnp.dot(quant, dec_w_ref[...],
                  preferred_element_type=jnp.float32) + dec_b_ref[...]
    dec_ref[...] = dec
    diff = dec - tpatch_ref[...]
    rec_blk = jnp.sum(diff * diff)

    @pl.when(i == 0)
    def _():
        rec_ref[0, 0] = 0.0
        vq_ref[0, 0] = 0.0

    rec_ref[0, 0] += rec_blk
    vq_ref[0, 0] += vq_blk

    @pl.when(i == nblk - 1)
    def _():
        loss_ref[0, 0] = rec_ref[0, 0] + 0.001 * vq_ref[0, 0]


def _patchify(x):
    B, C, H, W = x.shape
    hp, wp = H // P, W // P
    x = x.reshape(B, C, hp, P, wp, P)
    x = x.transpose(0, 2, 4, 1, 3, 5)
    return x.reshape(B * hp * wp, C * P * P), hp, wp


def kernel(input, target, enc_W, enc_b, codebook, dec_W, dec_b):
    B, C, H, W = input.shape
    patches, hp, wp = _patchify(input)
    tpatches, _, _ = _patchify(target)
    N = patches.shape[0]
    F = patches.shape[1]
    nblk = N // TBLK

    grid_spec = pl.GridSpec(
        grid=(nblk,),
        in_specs=[
            pl.BlockSpec((TBLK, F), lambda i: (i, 0)),
            pl.BlockSpec((TBLK, F), lambda i: (i, 0)),
            pl.BlockSpec((F, D), lambda i: (0, 0)),
            pl.BlockSpec((1, D), lambda i: (0, 0)),
            pl.BlockSpec((D, M), lambda i: (0, 0)),
            pl.BlockSpec((M, D), lambda i: (0, 0)),
            pl.BlockSpec((D, F), lambda i: (0, 0)),
            pl.BlockSpec((1, F), lambda i: (0, 0)),
        ],
        out_specs=[
            pl.BlockSpec((TBLK, F), lambda i: (i, 0)),
            pl.BlockSpec((1, 1), lambda i: (0, 0), memory_space=pltpu.SMEM),
            pl.BlockSpec((1, 1), lambda i: (0, 0), memory_space=pltpu.SMEM),
            pl.BlockSpec((1, 1), lambda i: (0, 0), memory_space=pltpu.SMEM),
        ],
    )

    dec, rec, vq, loss = pl.pallas_call(
        functools.partial(_vq_body, nblk),
        grid_spec=grid_spec,
        out_shape=[
            jax.ShapeDtypeStruct((N, F), jnp.float32),
            jax.ShapeDtypeStruct((1, 1), jnp.float32),
            jax.ShapeDtypeStruct((1, 1), jnp.float32),
            jax.ShapeDtypeStruct((1, 1), jnp.float32),
        ],
    )(patches, tpatches, enc_W, enc_b.reshape(1, D), codebook.T, codebook,
      dec_W, dec_b.reshape(1, F))

    y = dec.reshape(B, hp, wp, C, P, P)
    y = y.transpose(0, 3, 1, 4, 2, 5)
    sample = y.reshape(B, C, H, W)
    return sample, rec[0, 0], vq[0, 0], loss[0, 0]


# speedup vs baseline: 3.0536x; 2.3723x over previous
import functools

import jax
import jax.numpy as jnp
from jax.experimental import pallas as pl
from jax.experimental.pallas import tpu as pltpu

P = 16
D = 32
M = 8192
RBLK = 4


def _vq_body(nb, ni, x_ref, t_ref, enc_w_ref, enc_b_ref, cbt_ref,
             cb_ref, dec_w_ref, dec_b_ref, out_ref, rec_ref, vq_ref, loss_ref):
    b = pl.program_id(0)
    i = pl.program_id(1)

    C, RH, W = x_ref.shape[1], x_ref.shape[2], x_ref.shape[3]
    wp = W // P
    T = RBLK * wp

    x = x_ref[0]
    xp = x.reshape(C, RBLK, P, wp, P)
    xp = xp.transpose(1, 3, 0, 2, 4).reshape(T, C * P * P)

    lat = jnp.dot(xp, enc_w_ref[...],
                  preferred_element_type=jnp.float32) + enc_b_ref[...]
    scores = jnp.dot(lat, cbt_ref[...], preferred_element_type=jnp.float32)
    ind = jnp.argmax(scores, axis=1).astype(jnp.int32)

    onehot = (jax.lax.broadcasted_iota(jnp.int32, (T, M), 1)
              == ind[:, None]).astype(jnp.float32)
    quant = jnp.dot(onehot, cb_ref[...], preferred_element_type=jnp.float32)

    num = jnp.sum(lat * quant, axis=1)
    ln = jnp.sqrt(jnp.sum(lat * lat, axis=1))
    qn = jnp.sqrt(jnp.sum(quant * quant, axis=1))
    cos = num / jnp.maximum(ln * qn, 1e-8)
    vq_blk = jnp.sum(1.0 - cos)

    dec = jnp.dot(quant, dec_w_ref[...],
                  preferred_element_type=jnp.float32) + dec_b_ref[...]
    y = dec.reshape(RBLK, wp, C, P, P)
    y = y.transpose(2, 0, 3, 1, 4).reshape(C, RBLK * P, W)
    out_ref[0] = y
    diff = y - t_ref[0]
    rec_blk = jnp.sum(diff * diff)

    @pl.when(jnp.logical_and(b == 0, i == 0))
    def _():
        rec_ref[0, 0] = 0.0
        vq_ref[0, 0] = 0.0

    rec_ref[0, 0] += rec_blk
    vq_ref[0, 0] += vq_blk

    @pl.when(jnp.logical_and(b == nb - 1, i == ni - 1))
    def _():
        loss_ref[0, 0] = rec_ref[0, 0] + 0.001 * vq_ref[0, 0]


def kernel(input, target, enc_W, enc_b, codebook, dec_W, dec_b):
    B, C, H, W = input.shape
    F = C * P * P
    ni = H // (RBLK * P)

    grid_spec = pl.GridSpec(
        grid=(B, ni),
        in_specs=[
            pl.BlockSpec((1, C, RBLK * P, W), lambda b, i: (b, 0, i, 0)),
            pl.BlockSpec((1, C, RBLK * P, W), lambda b, i: (b, 0, i, 0)),
            pl.BlockSpec((F, D), lambda b, i: (0, 0)),
            pl.BlockSpec((1, D), lambda b, i: (0, 0)),
            pl.BlockSpec((D, M), lambda b, i: (0, 0)),
            pl.BlockSpec((M, D), lambda b, i: (0, 0)),
            pl.BlockSpec((D, F), lambda b, i: (0, 0)),
            pl.BlockSpec((1, F), lambda b, i: (0, 0)),
        ],
        out_specs=[
            pl.BlockSpec((1, C, RBLK * P, W), lambda b, i: (b, 0, i, 0)),
            pl.BlockSpec((1, 1), lambda b, i: (0, 0), memory_space=pltpu.SMEM),
            pl.BlockSpec((1, 1), lambda b, i: (0, 0), memory_space=pltpu.SMEM),
            pl.BlockSpec((1, 1), lambda b, i: (0, 0), memory_space=pltpu.SMEM),
        ],
    )

    sample, rec, vq, loss = pl.pallas_call(
        functools.partial(_vq_body, B, ni),
        grid_spec=grid_spec,
        out_shape=[
            jax.ShapeDtypeStruct((B, C, H, W), jnp.float32),
            jax.ShapeDtypeStruct((1, 1), jnp.float32),
            jax.ShapeDtypeStruct((1, 1), jnp.float32),
            jax.ShapeDtypeStruct((1, 1), jnp.float32),
        ],
    )(input, target, enc_W, enc_b.reshape(1, D), codebook.T, codebook,
      dec_W, dec_b.reshape(1, F))

    return sample, rec[0, 0], vq[0, 0], loss[0, 0]


# speedup vs baseline: 3.3619x; 1.1010x over previous
import functools

import jax
import jax.numpy as jnp
from jax import lax
from jax.experimental import pallas as pl
from jax.experimental.pallas import tpu as pltpu
from jax.experimental.pallas import tpu_sc as plsc

P = 16
D = 32
M = 8192
TBLK = 128


def _cbfull_body(cb_ref, dec_w_ref, dec_b_ref, out_ref):
    out_ref[...] = jnp.dot(cb_ref[...], dec_w_ref[...],
                           preferred_element_type=jnp.float32) + dec_b_ref[...]


def _codebook_full(codebook, dec_W, dec_b_row, F):
    nblk = 8
    rb = M // nblk
    return pl.pallas_call(
        _cbfull_body,
        grid=(nblk,),
        in_specs=[pl.BlockSpec((rb, D), lambda i: (i, 0)),
                  pl.BlockSpec((D, F), lambda i: (0, 0)),
                  pl.BlockSpec((1, F), lambda i: (0, 0))],
        out_specs=pl.BlockSpec((rb, F), lambda i: (i, 0)),
        out_shape=jax.ShapeDtypeStruct((M, F), jnp.float32),
    )(codebook, dec_W, dec_b_row)


def _make_patchify_sc(B, C, H, W, F):
    wp = W // P
    hp = H // P
    nrow = B * hp
    info = plsc.get_sparse_core_info()
    NW = info.num_cores * info.num_subcores
    rows_per_w = nrow // NW
    mesh = plsc.VectorSubcoreMesh(core_axis_name="c", subcore_axis_name="s")

    @functools.partial(
        pl.kernel, mesh=mesh,
        out_type=jax.ShapeDtypeStruct((B * hp * wp, F), jnp.float32),
        scratch_types=[pltpu.VMEM((C, P, W), jnp.float32),
                       pltpu.VMEM((wp, F), jnp.float32)],
    )
    def patchify(x_hbm, patches_hbm, slab_v, patch_v):
        wid = lax.axis_index("s") * info.num_cores + lax.axis_index("c")
        for k in range(rows_per_w):
            row = wid * rows_per_w + k
            b = row // hp
            i = row % hp
            for c in range(C):
                pltpu.sync_copy(x_hbm.at[b, c, pl.ds(i * P, P)],
                                slab_v.at[c])

            def rearrange(j, _):
                for c in range(C):
                    for pr in range(P):
                        patch_v[j, pl.ds((c * P + pr) * P, P)] = (
                            slab_v[c, pr, pl.ds(j * P, P)])
                return 0

            lax.fori_loop(0, wp, rearrange, 0)
            pltpu.sync_copy(patch_v, patches_hbm.at[pl.ds(row * wp, wp)])

    return patchify


def _assign_body(nblk, p_ref, enc_w_ref, enc_b_ref, cbt_ref,
                 ind_ref, vq_ref):
    s = pl.program_id(0)
    lat = jnp.dot(p_ref[...], enc_w_ref[...],
                  preferred_element_type=jnp.float32) + enc_b_ref[...]
    scores = jnp.dot(lat, cbt_ref[...], preferred_element_type=jnp.float32)
    ind = jnp.argmax(scores, axis=1).astype(jnp.int32)
    best = jnp.max(scores, axis=1)

    onehot = (jax.lax.broadcasted_iota(jnp.int32, (TBLK, M), 1)
              == ind[:, None]).astype(jnp.float32)
    cbn2 = jnp.sum(cbt_ref[...] * cbt_ref[...], axis=0)[None, :]
    qn2 = jnp.sum(onehot * cbn2, axis=1)
    ln2 = jnp.sum(lat * lat, axis=1)
    cos = best / jnp.maximum(jnp.sqrt(ln2) * jnp.sqrt(qn2), 1e-8)
    vq_blk = jnp.sum(1.0 - cos)

    ind_ref[...] = ind.reshape(1, 1, TBLK)

    @pl.when(s == 0)
    def _():
        vq_ref[0, 0] = 0.0

    vq_ref[0, 0] += vq_blk


def _assign(patches, enc_W, enc_b_row, codebook_T, N, F):
    nblk = N // TBLK
    ind, vq = pl.pallas_call(
        functools.partial(_assign_body, nblk),
        grid=(nblk,),
        in_specs=[pl.BlockSpec((TBLK, F), lambda s: (s, 0)),
                  pl.BlockSpec((F, D), lambda s: (0, 0)),
                  pl.BlockSpec((1, D), lambda s: (0, 0)),
                  pl.BlockSpec((D, M), lambda s: (0, 0))],
        out_specs=[pl.BlockSpec((1, 1, TBLK), lambda s: (s, 0, 0)),
                   pl.BlockSpec((1, 1), lambda s: (0, 0),
                                memory_space=pltpu.SMEM)],
        out_shape=[jax.ShapeDtypeStruct((nblk, 1, TBLK), jnp.int32),
                   jax.ShapeDtypeStruct((1, 1), jnp.float32)],
    )(patches, enc_W, enc_b_row, codebook_T)
    return ind.reshape(N), vq


def _make_decode_sc(B, C, H, W, F):
    wp = W // P
    hp = H // P
    nrow = B * hp
    info = plsc.get_sparse_core_info()
    NW = info.num_cores * info.num_subcores
    rows_per_w = nrow // NW
    mesh = plsc.VectorSubcoreMesh(core_axis_name="c", subcore_axis_name="s")

    @functools.partial(
        pl.kernel, mesh=mesh,
        out_type=jax.ShapeDtypeStruct((B, C, H, W), jnp.float32),
        scratch_types=[pltpu.VMEM((wp,), jnp.int32),
                       pltpu.VMEM((wp, F), jnp.float32),
                       pltpu.VMEM((C, P, W), jnp.float32),
                       pltpu.SemaphoreType.DMA],
    )
    def decode(cbfull_hbm, ind_hbm, out_hbm, idx_v, rows_v, slab_v, sem):
        wid = lax.axis_index("s") * info.num_cores + lax.axis_index("c")
        for k in range(rows_per_w):
            row = wid * rows_per_w + k
            b = row // hp
            i = row % hp
            pltpu.sync_copy(ind_hbm.at[pl.ds(row * wp, wp)], idx_v)
            pltpu.async_copy(cbfull_hbm.at[idx_v], rows_v, sem).wait()

            def rearrange(j, _):
                for c in range(C):
                    for pr in range(P):
                        slab_v[c, pr, pl.ds(j * P, P)] = (
                            rows_v[j, pl.ds((c * P + pr) * P, P)])
                return 0

            lax.fori_loop(0, wp, rearrange, 0)
            for c in range(C):
                pltpu.sync_copy(slab_v.at[c],
                                out_hbm.at[b, c, pl.ds(i * P, P)])

    return decode


def _rec_body(nb, s_ref, t_ref, vq_ref, rec_ref, loss_ref):
    b = pl.program_id(0)
    diff = s_ref[...] - t_ref[...]
    blk = jnp.sum(diff * diff)

    @pl.when(b == 0)
    def _():
        rec_ref[0, 0] = 0.0

    rec_ref[0, 0] += blk

    @pl.when(b == nb - 1)
    def _():
        loss_ref[0, 0] = rec_ref[0, 0] + 0.001 * vq_ref[0, 0]


def _rec_loss(sample, target, vq, B, C, H, W):
    return pl.pallas_call(
        functools.partial(_rec_body, B),
        grid=(B,),
        in_specs=[pl.BlockSpec((1, C, H, W), lambda b: (b, 0, 0, 0)),
                  pl.BlockSpec((1, C, H, W), lambda b: (b, 0, 0, 0)),
                  pl.BlockSpec((1, 1), lambda b: (0, 0),
                               memory_space=pltpu.SMEM)],
        out_specs=[pl.BlockSpec((1, 1), lambda b: (0, 0),
                                memory_space=pltpu.SMEM),
                   pl.BlockSpec((1, 1), lambda b: (0, 0),
                                memory_space=pltpu.SMEM)],
        out_shape=[jax.ShapeDtypeStruct((1, 1), jnp.float32),
                   jax.ShapeDtypeStruct((1, 1), jnp.float32)],
    )(sample, target, vq)


def kernel(input, target, enc_W, enc_b, codebook, dec_W, dec_b):
    B, C, H, W = input.shape
    F = C * P * P
    N = B * (H // P) * (W // P)

    cbfull = _codebook_full(codebook, dec_W, dec_b.reshape(1, F), F)
    patches = _make_patchify_sc(B, C, H, W, F)(input)
    ind, vq = _assign(patches, enc_W, enc_b.reshape(1, D), codebook.T, N, F)
    sample = _make_decode_sc(B, C, H, W, F)(cbfull, ind)
    rec, loss = _rec_loss(sample, target, vq, B, C, H, W)

    return sample, rec[0, 0], vq[0, 0], loss[0, 0]


# speedup vs baseline: 3.7050x; 1.1020x over previous
import functools

import jax
import jax.numpy as jnp
from jax import lax
from jax.experimental import pallas as pl
from jax.experimental.pallas import tpu as pltpu
from jax.experimental.pallas import tpu_sc as plsc

P = 16
D = 32
M = 8192
TBLK = 128


def _cbfull_body(cb_ref, dec_w_ref, dec_b_ref, out_ref):
    out_ref[...] = jnp.dot(cb_ref[...], dec_w_ref[...],
                           preferred_element_type=jnp.float32) + dec_b_ref[...]


def _codebook_full(codebook, dec_W, dec_b_row, F):
    nblk = 8
    rb = M // nblk
    return pl.pallas_call(
        _cbfull_body,
        grid=(nblk,),
        in_specs=[pl.BlockSpec((rb, D), lambda i: (i, 0)),
                  pl.BlockSpec((D, F), lambda i: (0, 0)),
                  pl.BlockSpec((1, F), lambda i: (0, 0))],
        out_specs=pl.BlockSpec((rb, F), lambda i: (i, 0)),
        out_shape=jax.ShapeDtypeStruct((M, F), jnp.float32),
    )(codebook, dec_W, dec_b_row)


def _make_patchify_sc(B, C, H, W, F, row0, nrow):
    wp = W // P
    hp = H // P
    info = plsc.get_sparse_core_info()
    NW = info.num_cores * info.num_subcores
    rows_per_w = nrow // NW
    mesh = plsc.VectorSubcoreMesh(core_axis_name="c", subcore_axis_name="s")

    @functools.partial(
        pl.kernel, mesh=mesh,
        out_type=jax.ShapeDtypeStruct((nrow * wp, F), jnp.float32),
        scratch_types=[pltpu.VMEM((C, P, W), jnp.float32),
                       pltpu.VMEM((wp, F), jnp.float32)],
    )
    def patchify(x_hbm, patches_hbm, slab_v, patch_v):
        wid = lax.axis_index("s") * info.num_cores + lax.axis_index("c")
        for k in range(rows_per_w):
            lrow = wid * rows_per_w + k
            row = row0 + lrow
            b = row // hp
            i = row % hp
            for c in range(C):
                pltpu.sync_copy(x_hbm.at[b, c, pl.ds(i * P, P)],
                                slab_v.at[c])

            def rearrange(j, _):
                for c in range(C):
                    for pr in range(P):
                        patch_v[j, pl.ds((c * P + pr) * P, P)] = (
                            slab_v[c, pr, pl.ds(j * P, P)])
                return 0

            lax.fori_loop(0, wp, rearrange, 0)
            pltpu.sync_copy(patch_v, patches_hbm.at[pl.ds(lrow * wp, wp)])

    return patchify


def _assign_body(nblk, p_ref, enc_w_ref, enc_b_ref, cbt_ref,
                 ind_ref, vq_ref):
    s = pl.program_id(0)
    lat = jnp.dot(p_ref[...], enc_w_ref[...],
                  preferred_element_type=jnp.float32) + enc_b_ref[...]
    scores = jnp.dot(lat, cbt_ref[...], preferred_element_type=jnp.float32)
    ind = jnp.argmax(scores, axis=1).astype(jnp.int32)
    best = jnp.max(scores, axis=1)

    onehot = (jax.lax.broadcasted_iota(jnp.int32, (TBLK, M), 1)
              == ind[:, None]).astype(jnp.float32)
    cbn2 = jnp.sum(cbt_ref[...] * cbt_ref[...], axis=0)[None, :]
    qn2 = jnp.sum(onehot * cbn2, axis=1)
    ln2 = jnp.sum(lat * lat, axis=1)
    cos = best / jnp.maximum(jnp.sqrt(ln2) * jnp.sqrt(qn2), 1e-8)
    vq_blk = jnp.sum(1.0 - cos)

    ind_ref[...] = ind.reshape(1, 1, TBLK)

    @pl.when(s == 0)
    def _():
        vq_ref[0, 0] = 0.0

    vq_ref[0, 0] += vq_blk


def _assign(patches, enc_W, enc_b_row, codebook_T, N, F):
    nblk = N // TBLK
    ind, vq = pl.pallas_call(
        functools.partial(_assign_body, nblk),
        grid=(nblk,),
        in_specs=[pl.BlockSpec((TBLK, F), lambda s: (s, 0)),
                  pl.BlockSpec((F, D), lambda s: (0, 0)),
                  pl.BlockSpec((1, D), lambda s: (0, 0)),
                  pl.BlockSpec((D, M), lambda s: (0, 0))],
        out_specs=[pl.BlockSpec((1, 1, TBLK), lambda s: (s, 0, 0)),
                   pl.BlockSpec((1, 1), lambda s: (0, 0),
                                memory_space=pltpu.SMEM)],
        out_shape=[jax.ShapeDtypeStruct((nblk, 1, TBLK), jnp.int32),
                   jax.ShapeDtypeStruct((1, 1), jnp.float32)],
    )(patches, enc_W, enc_b_row, codebook_T)
    return ind.reshape(N), vq


def _make_decode_sc(B, C, H, W, F):
    wp = W // P
    hp = H // P
    nrow = B * hp
    info = plsc.get_sparse_core_info()
    NW = info.num_cores * info.num_subcores
    rows_per_w = nrow // NW
    mesh = plsc.VectorSubcoreMesh(core_axis_name="c", subcore_axis_name="s")

    @functools.partial(
        pl.kernel, mesh=mesh,
        out_type=jax.ShapeDtypeStruct((B, C, H, W), jnp.float32),
        scratch_types=[pltpu.VMEM((wp,), jnp.int32),
                       pltpu.VMEM((wp, F), jnp.float32),
                       pltpu.VMEM((C, P, W), jnp.float32),
                       pltpu.SemaphoreType.DMA],
    )
    def decode(cbfull_hbm, ind_hbm, out_hbm, idx_v, rows_v, slab_v, sem):
        wid = lax.axis_index("s") * info.num_cores + lax.axis_index("c")
        for k in range(rows_per_w):
            row = wid * rows_per_w + k
            b = row // hp
            i = row % hp
            pltpu.sync_copy(ind_hbm.at[pl.ds(row * wp, wp)], idx_v)
            pltpu.async_copy(cbfull_hbm.at[idx_v], rows_v, sem).wait()

            def rearrange(j, _):
                for c in range(C):
                    for pr in range(P):
                        slab_v[c, pr, pl.ds(j * P, P)] = (
                            rows_v[j, pl.ds((c * P + pr) * P, P)])
                return 0

            lax.fori_loop(0, wp, rearrange, 0)
            for c in range(C):
                pltpu.sync_copy(slab_v.at[c],
                                out_hbm.at[b, c, pl.ds(i * P, P)])

    return decode


def _rec_body(nb, s_ref, t_ref, vq_ref, rec_ref, loss_ref):
    b = pl.program_id(0)
    diff = s_ref[...] - t_ref[...]
    blk = jnp.sum(diff * diff)

    @pl.when(b == 0)
    def _():
        rec_ref[0, 0] = 0.0

    rec_ref[0, 0] += blk

    @pl.when(b == nb - 1)
    def _():
        loss_ref[0, 0] = rec_ref[0, 0] + 0.001 * vq_ref[0, 0]


def _rec_loss(sample, target, vq, B, C, H, W):
    return pl.pallas_call(
        functools.partial(_rec_body, B),
        grid=(B,),
        in_specs=[pl.BlockSpec((1, C, H, W), lambda b: (b, 0, 0, 0)),
                  pl.BlockSpec((1, C, H, W), lambda b: (b, 0, 0, 0)),
                  pl.BlockSpec((1, 1), lambda b: (0, 0),
                               memory_space=pltpu.SMEM)],
        out_specs=[pl.BlockSpec((1, 1), lambda b: (0, 0),
                                memory_space=pltpu.SMEM),
                   pl.BlockSpec((1, 1), lambda b: (0, 0),
                                memory_space=pltpu.SMEM)],
        out_shape=[jax.ShapeDtypeStruct((1, 1), jnp.float32),
                   jax.ShapeDtypeStruct((1, 1), jnp.float32)],
    )(sample, target, vq)


def kernel(input, target, enc_W, enc_b, codebook, dec_W, dec_b):
    B, C, H, W = input.shape
    F = C * P * P
    hp = H // P
    nrow = B * hp
    half = nrow // 2

    cbfull = _codebook_full(codebook, dec_W, dec_b.reshape(1, F), F)
    enc_b_row = enc_b.reshape(1, D)
    cbT = codebook.T

    patches_a = _make_patchify_sc(B, C, H, W, F, 0, half)(input)
    patches_b = _make_patchify_sc(B, C, H, W, F, half, half)(input)
    N2 = half * (W // P)
    ind_a, vq_a = _assign(patches_a, enc_W, enc_b_row, cbT, N2, F)
    ind_b, vq_b = _assign(patches_b, enc_W, enc_b_row, cbT, N2, F)
    vq = vq_a + vq_b
    ind = jnp.concatenate([ind_a, ind_b])
    sample = _make_decode_sc(B, C, H, W, F)(cbfull, ind)
    rec, loss = _rec_loss(sample, target, vq, B, C, H, W)

    return sample, rec[0, 0], vq[0, 0], loss[0, 0]


# speedup vs baseline: 4.0234x; 1.0859x over previous
import functools

import jax
import jax.numpy as jnp
from jax import lax
from jax.experimental import pallas as pl
from jax.experimental.pallas import tpu as pltpu
from jax.experimental.pallas import tpu_sc as plsc

P = 16
D = 32
M = 8192
TBLK = 128


def _cbfull_body(cb_ref, dec_w_ref, dec_b_ref, out_ref):
    out_ref[...] = jnp.dot(cb_ref[...], dec_w_ref[...],
                           preferred_element_type=jnp.float32) + dec_b_ref[...]


def _codebook_full(codebook, dec_W, dec_b_row, F):
    nblk = 8
    rb = M // nblk
    return pl.pallas_call(
        _cbfull_body,
        grid=(nblk,),
        in_specs=[pl.BlockSpec((rb, D), lambda i: (i, 0)),
                  pl.BlockSpec((D, F), lambda i: (0, 0)),
                  pl.BlockSpec((1, F), lambda i: (0, 0))],
        out_specs=pl.BlockSpec((rb, F), lambda i: (i, 0)),
        out_shape=jax.ShapeDtypeStruct((M, F), jnp.float32),
    )(codebook, dec_W, dec_b_row)


def _make_patchify_sc(B, C, H, W, F, row0, nrow):
    wp = W // P
    hp = H // P
    info = plsc.get_sparse_core_info()
    NW = info.num_cores * info.num_subcores
    rows_per_w = nrow // NW
    mesh = plsc.VectorSubcoreMesh(core_axis_name="c", subcore_axis_name="s")

    @functools.partial(
        pl.kernel, mesh=mesh,
        out_type=jax.ShapeDtypeStruct((nrow * wp, F), jnp.float32),
        scratch_types=[pltpu.VMEM((2, C, P, W), jnp.float32),
                       pltpu.VMEM((2, wp, F), jnp.float32),
                       pltpu.SemaphoreType.DMA,
                       pltpu.SemaphoreType.DMA,
                       pltpu.SemaphoreType.DMA,
                       pltpu.SemaphoreType.DMA],
    )
    def patchify(x_hbm, patches_hbm, slab_v, patch_v, s0, s1, w0, w1):
        wid = lax.axis_index("s") * info.num_cores + lax.axis_index("c")
        rsem = (s0, s1)
        wsem = (w0, w1)

        def fire_reads(k, buf):
            row = row0 + wid * rows_per_w + k
            b = row // hp
            i = row % hp
            return [pltpu.async_copy(x_hbm.at[b, c, pl.ds(i * P, P)],
                                     slab_v.at[buf, c], rsem[buf])
                    for c in range(C)]

        reads = fire_reads(0, 0)
        writes = [None, None]
        for k in range(rows_per_w):
            cur = k & 1
            nxt = 1 - cur
            if k + 1 < rows_per_w:
                nreads = fire_reads(k + 1, nxt)
            for d in reads:
                d.wait()
            if writes[cur] is not None:
                writes[cur].wait()

            def rearrange(j, _):
                for c in range(C):
                    for pr in range(P):
                        patch_v[cur, j, pl.ds((c * P + pr) * P, P)] = (
                            slab_v[cur, c, pr, pl.ds(j * P, P)])
                return 0

            lax.fori_loop(0, wp, rearrange, 0)
            lrow = wid * rows_per_w + k
            writes[cur] = pltpu.async_copy(
                patch_v.at[cur], patches_hbm.at[pl.ds(lrow * wp, wp)],
                wsem[cur])
            if k + 1 < rows_per_w:
                reads = nreads
        for d in writes:
            if d is not None:
                d.wait()

    return patchify


def _assign_body(nblk, p_ref, enc_w_ref, enc_b_ref, cbt_ref,
                 ind_ref, vq_ref):
    s = pl.program_id(0)
    lat = jnp.dot(p_ref[...], enc_w_ref[...],
                  preferred_element_type=jnp.float32) + enc_b_ref[...]
    scores = jnp.dot(lat, cbt_ref[...], preferred_element_type=jnp.float32)
    ind = jnp.argmax(scores, axis=1).astype(jnp.int32)
    best = jnp.max(scores, axis=1)

    onehot = (jax.lax.broadcasted_iota(jnp.int32, (TBLK, M), 1)
              == ind[:, None]).astype(jnp.float32)
    cbn2 = jnp.sum(cbt_ref[...] * cbt_ref[...], axis=0)[None, :]
    qn2 = jnp.sum(onehot * cbn2, axis=1)
    ln2 = jnp.sum(lat * lat, axis=1)
    cos = best / jnp.maximum(jnp.sqrt(ln2) * jnp.sqrt(qn2), 1e-8)
    vq_blk = jnp.sum(1.0 - cos)

    ind_ref[...] = ind.reshape(1, 1, TBLK)

    @pl.when(s == 0)
    def _():
        vq_ref[0, 0] = 0.0

    vq_ref[0, 0] += vq_blk


def _assign(patches, enc_W, enc_b_row, codebook_T, N, F):
    nblk = N // TBLK
    ind, vq = pl.pallas_call(
        functools.partial(_assign_body, nblk),
        grid=(nblk,),
        in_specs=[pl.BlockSpec((TBLK, F), lambda s: (s, 0)),
                  pl.BlockSpec((F, D), lambda s: (0, 0)),
                  pl.BlockSpec((1, D), lambda s: (0, 0)),
                  pl.BlockSpec((D, M), lambda s: (0, 0))],
        out_specs=[pl.BlockSpec((1, 1, TBLK), lambda s: (s, 0, 0)),
                   pl.BlockSpec((1, 1), lambda s: (0, 0),
                                memory_space=pltpu.SMEM)],
        out_shape=[jax.ShapeDtypeStruct((nblk, 1, TBLK), jnp.int32),
                   jax.ShapeDtypeStruct((1, 1), jnp.float32)],
    )(patches, enc_W, enc_b_row, codebook_T)
    return ind.reshape(N), vq


def _make_decode_sc(B, C, H, W, F):
    wp = W // P
    hp = H // P
    nrow = B * hp
    info = plsc.get_sparse_core_info()
    NW = info.num_cores * info.num_subcores
    rows_per_w = nrow // NW
    mesh = plsc.VectorSubcoreMesh(core_axis_name="c", subcore_axis_name="s")

    @functools.partial(
        pl.kernel, mesh=mesh,
        out_type=jax.ShapeDtypeStruct((B, C, H, W), jnp.float32),
        scratch_types=[pltpu.VMEM((2, wp), jnp.int32),
                       pltpu.VMEM((2, wp, F), jnp.float32),
                       pltpu.VMEM((2, C, P, W), jnp.float32),
                       pltpu.SemaphoreType.DMA,
                       pltpu.SemaphoreType.DMA,
                       pltpu.SemaphoreType.DMA,
                       pltpu.SemaphoreType.DMA],
    )
    def decode(cbfull_hbm, ind_hbm, out_hbm, idx_v, rows_v, slab_v,
               g0, g1, w0, w1):
        wid = lax.axis_index("s") * info.num_cores + lax.axis_index("c")
        gsem = (g0, g1)
        wsem = (w0, w1)

        def fire_gather(k, buf):
            row = wid * rows_per_w + k
            pltpu.sync_copy(ind_hbm.at[pl.ds(row * wp, wp)], idx_v.at[buf])
            return pltpu.async_copy(cbfull_hbm.at[idx_v.at[buf]],
                                    rows_v.at[buf], gsem[buf])

        gd = fire_gather(0, 0)
        writes = [None, None]
        for k in range(rows_per_w):
            cur = k & 1
            nxt = 1 - cur
            if k + 1 < rows_per_w:
                ngd = fire_gather(k + 1, nxt)
            gd.wait()
            if writes[cur] is not None:
                for d in writes[cur]:
                    d.wait()

            def rearrange(j, _):
                for c in range(C):
                    for pr in range(P):
                        slab_v[cur, c, pr, pl.ds(j * P, P)] = (
                            rows_v[cur, j, pl.ds((c * P + pr) * P, P)])
                return 0

            lax.fori_loop(0, wp, rearrange, 0)
            row = wid * rows_per_w + k
            b = row // hp
            i = row % hp
            writes[cur] = [pltpu.async_copy(
                slab_v.at[cur, c], out_hbm.at[b, c, pl.ds(i * P, P)],
                wsem[cur]) for c in range(C)]
            if k + 1 < rows_per_w:
                gd = ngd
        for ds_ in writes:
            if ds_ is not None:
                for d in ds_:
                    d.wait()

    return decode


def _rec_body(nb, s_ref, t_ref, vq_ref, rec_ref, loss_ref):
    b = pl.program_id(0)
    diff = s_ref[...] - t_ref[...]
    blk = jnp.sum(diff * diff)

    @pl.when(b == 0)
    def _():
        rec_ref[0, 0] = 0.0

    rec_ref[0, 0] += blk

    @pl.when(b == nb - 1)
    def _():
        loss_ref[0, 0] = rec_ref[0, 0] + 0.001 * vq_ref[0, 0]


def _rec_loss(sample, target, vq, B, C, H, W):
    return pl.pallas_call(
        functools.partial(_rec_body, B),
        grid=(B,),
        in_specs=[pl.BlockSpec((1, C, H, W), lambda b: (b, 0, 0, 0)),
                  pl.BlockSpec((1, C, H, W), lambda b: (b, 0, 0, 0)),
                  pl.BlockSpec((1, 1), lambda b: (0, 0),
                               memory_space=pltpu.SMEM)],
        out_specs=[pl.BlockSpec((1, 1), lambda b: (0, 0),
                                memory_space=pltpu.SMEM),
                   pl.BlockSpec((1, 1), lambda b: (0, 0),
                                memory_space=pltpu.SMEM)],
        out_shape=[jax.ShapeDtypeStruct((1, 1), jnp.float32),
                   jax.ShapeDtypeStruct((1, 1), jnp.float32)],
    )(sample, target, vq)


def kernel(input, target, enc_W, enc_b, codebook, dec_W, dec_b):
    B, C, H, W = input.shape
    F = C * P * P
    hp = H // P
    nrow = B * hp
    half = nrow // 2

    cbfull = _codebook_full(codebook, dec_W, dec_b.reshape(1, F), F)
    enc_b_row = enc_b.reshape(1, D)
    cbT = codebook.T

    patches_a = _make_patchify_sc(B, C, H, W, F, 0, half)(input)
    patches_b = _make_patchify_sc(B, C, H, W, F, half, half)(input)
    N2 = half * (W // P)
    ind_a, vq_a = _assign(patches_a, enc_W, enc_b_row, cbT, N2, F)
    ind_b, vq_b = _assign(patches_b, enc_W, enc_b_row, cbT, N2, F)
    vq = vq_a + vq_b
    ind = jnp.concatenate([ind_a, ind_b])
    sample = _make_decode_sc(B, C, H, W, F)(cbfull, ind)
    rec, loss = _rec_loss(sample, target, vq, B, C, H, W)

    return sample, rec[0, 0], vq[0, 0], loss[0, 0]
